# baseline (speedup 1.0000x reference)
import functools

import jax
import jax.numpy as jnp
from jax import lax
from jax.experimental import pallas as pl
from jax.experimental.pallas import tpu as pltpu
from jax.experimental.pallas import tpu_sc as plsc

_B, _L, _D = 1024, 200, 32
_NC, _NS = 2, 16
_NW = _NC * _NS
_RPW = _B // _NW
_T = 256
_PAD = 208
_NCHUNK = _PAD // 16


def _table_kernel(w1_ref, b1_ref, w2_ref, b2_ref, t2_ref, t1_ref):
    i = pl.program_id(0)

    @pl.when(i == 0)
    def _():
        a = lax.broadcasted_iota(jnp.int32, (_T, 1), 0).astype(jnp.float32)
        h = jnp.maximum(a * w1_ref[...] + b1_ref[...], 0.0)
        t1_ref[...] = (
            jnp.dot(h, w2_ref[...], preferred_element_type=jnp.float32)
            + b2_ref[...]
        )

    t1 = t1_ref[...]
    for j in range(16):
        row = t1_ref[pl.ds(i * 16 + j, 1), :]
        t2_ref[j * _T:(j + 1) * _T, :] = t1 + row


def _build_table(W1, b1, W2, b2):
    return pl.pallas_call(
        _table_kernel,
        grid=(16,),
        in_specs=[
            pl.BlockSpec((1, _D), lambda i: (0, 0)),
            pl.BlockSpec((1, _D), lambda i: (0, 0)),
            pl.BlockSpec((_D, _D), lambda i: (0, 0)),
            pl.BlockSpec((1, _D), lambda i: (0, 0)),
        ],
        out_specs=pl.BlockSpec((16 * _T, _D), lambda i: (i, 0)),
        out_shape=jax.ShapeDtypeStruct((_T * _T, _D), jnp.float32),
        scratch_shapes=[pltpu.VMEM((_T, _D), jnp.float32)],
    )(W1, b1.reshape(1, _D), W2, b2.reshape(1, _D))


@functools.partial(
    pl.kernel,
    out_type=(
        jax.ShapeDtypeStruct((_B, _L, _D), jnp.float32),
        jax.ShapeDtypeStruct((_B, _L, _D), jnp.float32),
    ),
    mesh=plsc.VectorSubcoreMesh(core_axis_name="c", subcore_axis_name="s"),
    compiler_params=pltpu.CompilerParams(
        needs_layout_passes=False, use_tc_tiling_on_sc=False),
    scratch_types=[
        pltpu.VMEM((2 * _PAD,), jnp.int32),
        pltpu.VMEM((2 * _PAD,), jnp.int32),
        pltpu.VMEM((1024,), jnp.int32),
        pltpu.VMEM((1024,), jnp.int32),
        pltpu.VMEM((2 * _PAD, _D), jnp.float32),
        pltpu.SemaphoreType.DMA,
    ],
)
def _sc_encode(src_hbm, dst_hbm, t2_hbm, out_src_hbm, out_dst_hbm,
               ids_v, pair_v, hs_v, hd_v, rows_v, sem):
    wid = lax.axis_index("s") * _NC + lax.axis_index("c")
    base = wid * _RPW
    zero16 = jnp.zeros((16,), jnp.int32)
    one16 = jnp.ones((16,), jnp.int32)

    def row_body(r, carry):
        row = base + r
        for k in range(64):
            hs_v[pl.ds(k * 16, 16)] = zero16
            hd_v[pl.ds(k * 16, 16)] = zero16
        ids_v[pl.ds(_L - 8, 16)] = zero16
        ids_v[pl.ds(_PAD + _L - 8, 16)] = zero16
        pltpu.sync_copy(src_hbm.at[row], ids_v.at[pl.ds(0, _L)])
        pltpu.sync_copy(dst_hbm.at[row], ids_v.at[pl.ds(_PAD, _L)])
        for c in range(_NCHUNK):
            plsc.addupdate_scatter(hs_v, [ids_v[pl.ds(c * 16, 16)]], one16)
        for c in range(_NCHUNK):
            plsc.addupdate_scatter(
                hd_v, [ids_v[pl.ds(_PAD + c * 16, 16)]], one16)
        for c in range(2 * _NCHUNK):
            idx = ids_v[pl.ds(c * 16, 16)]
            c1 = plsc.load_gather(hs_v, [idx])
            c2 = plsc.load_gather(hd_v, [idx])
            pair = c1 * _T + c2
            pair = jnp.where(idx == 0, zero16, pair)
            pair_v[pl.ds(c * 16, 16)] = pair
        copies = []
        for g in range(4):
            copies.append(pltpu.async_copy(
                t2_hbm.at[pair_v.at[pl.ds(g * 104, 104)]],
                rows_v.at[pl.ds(g * 104, 104)], sem))
        for cp in copies:
            cp.wait()
        pltpu.sync_copy(rows_v.at[pl.ds(0, _L)], out_src_hbm.at[row])
        pltpu.sync_copy(rows_v.at[pl.ds(_PAD, _L)], out_dst_hbm.at[row])
        return carry

    lax.fori_loop(0, _RPW, row_body, 0)


def kernel(src_ids, dst_ids, W1, b1, W2, b2):
    t2 = _build_table(W1, b1, W2, b2)
    out_src, out_dst = _sc_encode(src_ids, dst_ids, t2)
    return out_src, out_dst
